# Initial kernel scaffold; baseline (speedup 1.0000x reference)
import functools

import jax
import jax.numpy as jnp
from jax import lax
from jax.experimental import pallas as pl
from jax.experimental.pallas import tpu as pltpu
from jax.experimental.pallas import tpu_sc as plsc

N = 10000
E = 320000
C_IN = 128
C_OUT = 128
KT = 3

NC = 2
NS = 16
NW = NC * NS
EPT = E // NW
CHUNK = 128
NFULL = EPT // CHUNK
REMC = EPT - NFULL * CHUNK

NODE_A = 624
NODE_B = N - 15 * NODE_A

_mesh = plsc.VectorSubcoreMesh(core_axis_name="c", subcore_axis_name="s")


def _node_slice_copy(s, copy_a, copy_b):
    @pl.when(s < NS - 1)
    def _():
        copy_a()

    @pl.when(s == NS - 1)
    def _():
        copy_b()


@functools.partial(
    pl.kernel,
    out_type=jax.ShapeDtypeStruct((NC, N), jnp.float32),
    mesh=_mesh,
    scratch_types=[
        pltpu.VMEM((CHUNK,), jnp.int32),
        pltpu.VMEM((REMC,), jnp.int32),
        pltpu.VMEM((CHUNK,), jnp.float32),
        pltpu.VMEM((REMC,), jnp.float32),
        pltpu.VMEM_SHARED((N,), jnp.float32),
    ],
)
def _deg_kernel(dst_hbm, ones_hbm, zeros_hbm, out_hbm,
                idx_v, idxr_v, ones_v, onesr_v, deg_sp):
    c = lax.axis_index("c")
    s = lax.axis_index("s")
    wid = c * NS + s
    ebase = wid * EPT

    pltpu.sync_copy(ones_hbm.at[pl.ds(0, CHUNK)], ones_v)
    pltpu.sync_copy(ones_hbm.at[pl.ds(0, REMC)], onesr_v)
    _node_slice_copy(
        s,
        lambda: pltpu.sync_copy(zeros_hbm.at[pl.ds(0, NODE_A)],
                                deg_sp.at[pl.ds(s * NODE_A, NODE_A)]),
        lambda: pltpu.sync_copy(zeros_hbm.at[pl.ds(0, NODE_B)],
                                deg_sp.at[pl.ds((NS - 1) * NODE_A, NODE_B)]),
    )
    plsc.subcore_barrier()

    @pl.loop(0, NFULL)
    def _(ci):
        pltpu.sync_copy(dst_hbm.at[pl.ds(ebase + ci * CHUNK, CHUNK)], idx_v)
        pltpu.sync_copy(ones_v, deg_sp.at[idx_v], add=True)

    pltpu.sync_copy(dst_hbm.at[pl.ds(ebase + NFULL * CHUNK, REMC)], idxr_v)
    pltpu.sync_copy(onesr_v, deg_sp.at[idxr_v], add=True)

    plsc.subcore_barrier()
    _node_slice_copy(
        s,
        lambda: pltpu.sync_copy(deg_sp.at[pl.ds(s * NODE_A, NODE_A)],
                                out_hbm.at[c, pl.ds(s * NODE_A, NODE_A)]),
        lambda: pltpu.sync_copy(deg_sp.at[pl.ds((NS - 1) * NODE_A, NODE_B)],
                                out_hbm.at[c, pl.ds((NS - 1) * NODE_A, NODE_B)]),
    )


@functools.partial(
    pl.kernel,
    out_type=jax.ShapeDtypeStruct((NC, N, C_OUT), jnp.float32),
    mesh=_mesh,
    scratch_types=[
        pltpu.VMEM((CHUNK,), jnp.int32),
        pltpu.VMEM((CHUNK,), jnp.int32),
        pltpu.VMEM((CHUNK,), jnp.int32),
        pltpu.VMEM((CHUNK,), jnp.int32),
        pltpu.VMEM((CHUNK, C_OUT), jnp.float32),
        pltpu.VMEM((CHUNK, C_OUT), jnp.float32),
        pltpu.VMEM((REMC,), jnp.int32),
        pltpu.VMEM((REMC,), jnp.int32),
        pltpu.VMEM((REMC, C_OUT), jnp.float32),
        pltpu.VMEM_SHARED((N, C_OUT), jnp.float32),
        pltpu.SemaphoreType.DMA,
        pltpu.SemaphoreType.DMA,
        pltpu.SemaphoreType.DMA,
        pltpu.SemaphoreType.DMA,
    ],
)
def _scatter_kernel(src_hbm, dst_hbm, g_hbm, zeros2_hbm, out_hbm,
                    si0, si1, di0, di1, rows0, rows1,
                    sir, dir_, rowsr, acc_sp, gsem0, gsem1, ssem0, ssem1):
    c = lax.axis_index("c")
    s = lax.axis_index("s")
    wid = c * NS + s
    ebase = wid * EPT

    _node_slice_copy(
        s,
        lambda: pltpu.sync_copy(zeros2_hbm.at[pl.ds(0, NODE_A), :],
                                acc_sp.at[pl.ds(s * NODE_A, NODE_A), :]),
        lambda: pltpu.sync_copy(zeros2_hbm.at[pl.ds(0, NODE_B), :],
                                acc_sp.at[pl.ds((NS - 1) * NODE_A, NODE_B), :]),
    )
    plsc.subcore_barrier()

    bufs = ((si0, di0, rows0, gsem0, ssem0),
            (si1, di1, rows1, gsem1, ssem1))

    def load_idx(ci, b):
        si, di, _, _, _ = bufs[b]
        off = ebase + ci * CHUNK
        pltpu.sync_copy(src_hbm.at[pl.ds(off, CHUNK)], si)
        pltpu.sync_copy(dst_hbm.at[pl.ds(off, CHUNK)], di)

    def start_gather(b):
        si, _, rows, gsem, _ = bufs[b]
        pltpu.async_copy(g_hbm.at[si], rows, gsem)

    load_idx(0, 0)
    start_gather(0)
    load_idx(1, 1)

    @pl.loop(0, NFULL // 2)
    def _(j):
        for b in range(2):
            si, di, rows, gsem, ssem = bufs[b]
            ci = 2 * j + b
            pltpu.make_async_copy(g_hbm.at[si], rows, gsem).wait()
            pltpu.async_copy(rows, acc_sp.at[di], ssem, add=True)

            @pl.when(ci + 1 < NFULL)
            def _():
                start_gather(1 - b)

            pltpu.make_async_copy(rows, acc_sp.at[di], ssem).wait()

            @pl.when(ci + 2 < NFULL)
            def _():
                load_idx(ci + 2, b)

    off = ebase + NFULL * CHUNK
    pltpu.sync_copy(src_hbm.at[pl.ds(off, REMC)], sir)
    pltpu.sync_copy(dst_hbm.at[pl.ds(off, REMC)], dir_)
    pltpu.async_copy(g_hbm.at[sir], rowsr, gsem0).wait()
    pltpu.sync_copy(rowsr, acc_sp.at[dir_], add=True)

    plsc.subcore_barrier()
    _node_slice_copy(
        s,
        lambda: pltpu.sync_copy(acc_sp.at[pl.ds(s * NODE_A, NODE_A), :],
                                out_hbm.at[c, pl.ds(s * NODE_A, NODE_A), :]),
        lambda: pltpu.sync_copy(acc_sp.at[pl.ds((NS - 1) * NODE_A, NODE_B), :],
                                out_hbm.at[c, pl.ds((NS - 1) * NODE_A, NODE_B), :]),
    )


_BLK = 1000


def _dense_body(x2_ref, degp_ref, wcat_ref, bg_ref, bt_ref,
                g_ref, base_ref, dinv_ref):
    hu = jnp.dot(x2_ref[...], wcat_ref[...],
                 preferred_element_type=jnp.float32)
    h = hu[:, :C_OUT]
    tmp = hu[:, C_OUT:]
    deg = degp_ref[:, 0:1] + degp_ref[:, 1:2] + 1.0
    dinv = lax.rsqrt(deg)
    g_ref[...] = h * dinv
    base_ref[...] = h * (dinv * dinv) + bg_ref[...] + tmp + bt_ref[...]
    dinv_ref[...] = dinv


def _dense_call(x2, degp_t, wcat, bg, bt):
    return pl.pallas_call(
        _dense_body,
        grid=(N // _BLK,),
        in_specs=[
            pl.BlockSpec((_BLK, C_IN * KT), lambda i: (i, 0)),
            pl.BlockSpec((_BLK, NC), lambda i: (i, 0)),
            pl.BlockSpec((C_IN * KT, 2 * C_OUT), lambda i: (0, 0)),
            pl.BlockSpec((1, C_OUT), lambda i: (0, 0)),
            pl.BlockSpec((1, C_OUT), lambda i: (0, 0)),
        ],
        out_specs=[
            pl.BlockSpec((_BLK, C_OUT), lambda i: (i, 0)),
            pl.BlockSpec((_BLK, C_OUT), lambda i: (i, 0)),
            pl.BlockSpec((_BLK, 1), lambda i: (i, 0)),
        ],
        out_shape=[
            jax.ShapeDtypeStruct((N, C_OUT), jnp.float32),
            jax.ShapeDtypeStruct((N, C_OUT), jnp.float32),
            jax.ShapeDtypeStruct((N, 1), jnp.float32),
        ],
    )(x2, degp_t, wcat, bg, bt)


def _combine_body(p0_ref, p1_ref, dinv_ref, base_ref, out_ref):
    out_ref[...] = (dinv_ref[...] * (p0_ref[...] + p1_ref[...])
                    + base_ref[...])


def _combine_call(p0, p1, dinv, base):
    return pl.pallas_call(
        _combine_body,
        grid=(N // _BLK,),
        in_specs=[
            pl.BlockSpec((_BLK, C_OUT), lambda i: (i, 0)),
            pl.BlockSpec((_BLK, C_OUT), lambda i: (i, 0)),
            pl.BlockSpec((_BLK, 1), lambda i: (i, 0)),
            pl.BlockSpec((_BLK, C_OUT), lambda i: (i, 0)),
        ],
        out_specs=pl.BlockSpec((_BLK, C_OUT), lambda i: (i, 0)),
        out_shape=jax.ShapeDtypeStruct((N, C_OUT), jnp.float32),
    )(p0, p1, dinv, base)


def kernel(x, edge_index, W_gcn, b_gcn, W_t, b_t):
    x2 = x.reshape(N, C_IN * KT)
    w3 = jnp.repeat(W_gcn, KT, axis=0)
    w2 = W_t.transpose(1, 2, 0).reshape(C_IN * KT, C_OUT)
    wcat = jnp.concatenate([w3, w2], axis=1)

    src = edge_index[0]
    dst = edge_index[1]

    ones1 = jnp.ones((CHUNK,), jnp.float32)
    zeros1 = jnp.zeros((NODE_B,), jnp.float32)
    zeros2 = jnp.zeros((NODE_B, C_OUT), jnp.float32)

    degp = _deg_kernel(dst, ones1, zeros1)
    g, base, dinv = _dense_call(x2, degp.T, wcat,
                                b_gcn.reshape(1, C_OUT),
                                b_t.reshape(1, C_OUT))
    part = _scatter_kernel(src, dst, g, zeros2)
    return _combine_call(part[0], part[1], dinv, base)


# speedup vs baseline: 25.2604x; 25.2604x over previous
import functools

import jax
import jax.numpy as jnp
from jax import lax
from jax.experimental import pallas as pl
from jax.experimental.pallas import tpu as pltpu
from jax.experimental.pallas import tpu_sc as plsc

N = 10000
E = 320000
C_IN = 128
C_OUT = 128
KT = 3

NC = 2
NS = 16
NW = NC * NS
EPT = E // NW
CHUNK = 128
NFULL = EPT // CHUNK
REMC = EPT - NFULL * CHUNK

NODE_A = 624
NODE_B = N - 15 * NODE_A

_mesh = plsc.VectorSubcoreMesh(core_axis_name="c", subcore_axis_name="s")


def _node_slice_copy(s, copy_a, copy_b):
    @pl.when(s < NS - 1)
    def _():
        copy_a()

    @pl.when(s == NS - 1)
    def _():
        copy_b()


@functools.partial(
    pl.kernel,
    out_type=jax.ShapeDtypeStruct((NC * N,), jnp.float32),
    mesh=_mesh,
    scratch_types=[
        pltpu.VMEM((CHUNK,), jnp.int32),
        pltpu.VMEM((REMC,), jnp.int32),
        pltpu.VMEM((CHUNK,), jnp.float32),
        pltpu.VMEM((REMC,), jnp.float32),
        pltpu.VMEM((NODE_B,), jnp.float32),
        pltpu.VMEM_SHARED((N,), jnp.float32),
    ],
)
def _deg_kernel(dst_hbm, ones_hbm, zeros_hbm, out_hbm,
                idx_v, idxr_v, ones_v, onesr_v, zbuf_v, deg_sp):
    c = lax.axis_index("c")
    s = lax.axis_index("s")
    wid = c * NS + s
    ebase = wid * EPT

    pltpu.sync_copy(ones_hbm.at[pl.ds(0, CHUNK)], ones_v)
    pltpu.sync_copy(ones_hbm.at[pl.ds(0, REMC)], onesr_v)
    pltpu.sync_copy(zeros_hbm, zbuf_v)
    _node_slice_copy(
        s,
        lambda: pltpu.sync_copy(zbuf_v.at[pl.ds(0, NODE_A)],
                                deg_sp.at[pl.ds(s * NODE_A, NODE_A)]),
        lambda: pltpu.sync_copy(zbuf_v,
                                deg_sp.at[pl.ds((NS - 1) * NODE_A, NODE_B)]),
    )
    plsc.subcore_barrier()

    @pl.loop(0, NFULL)
    def _(ci):
        pltpu.sync_copy(dst_hbm.at[pl.ds(ebase + ci * CHUNK, CHUNK)], idx_v)
        pltpu.sync_copy(ones_v, deg_sp.at[idx_v], add=True)

    pltpu.sync_copy(dst_hbm.at[pl.ds(ebase + NFULL * CHUNK, REMC)], idxr_v)
    pltpu.sync_copy(onesr_v, deg_sp.at[idxr_v], add=True)

    plsc.subcore_barrier()

    def _wr_a():
        pltpu.sync_copy(deg_sp.at[pl.ds(s * NODE_A, NODE_A)],
                        zbuf_v.at[pl.ds(0, NODE_A)])
        pltpu.sync_copy(zbuf_v.at[pl.ds(0, NODE_A)],
                        out_hbm.at[pl.ds(c * N + s * NODE_A, NODE_A)])

    def _wr_b():
        pltpu.sync_copy(deg_sp.at[pl.ds((NS - 1) * NODE_A, NODE_B)], zbuf_v)
        pltpu.sync_copy(zbuf_v,
                        out_hbm.at[pl.ds(c * N + (NS - 1) * NODE_A, NODE_B)])

    _node_slice_copy(s, _wr_a, _wr_b)


@functools.partial(
    pl.kernel,
    out_type=jax.ShapeDtypeStruct((NC, N, C_OUT), jnp.float32),
    mesh=_mesh,
    scratch_types=[
        pltpu.VMEM((CHUNK,), jnp.int32),
        pltpu.VMEM((CHUNK,), jnp.int32),
        pltpu.VMEM((CHUNK,), jnp.int32),
        pltpu.VMEM((CHUNK,), jnp.int32),
        pltpu.VMEM((CHUNK, C_OUT), jnp.float32),
        pltpu.VMEM((CHUNK, C_OUT), jnp.float32),
        pltpu.VMEM((REMC,), jnp.int32),
        pltpu.VMEM((REMC,), jnp.int32),
        pltpu.VMEM((REMC, C_OUT), jnp.float32),
        pltpu.VMEM_SHARED((N, C_OUT), jnp.float32),
        pltpu.SemaphoreType.DMA,
        pltpu.SemaphoreType.DMA,
        pltpu.SemaphoreType.DMA,
        pltpu.SemaphoreType.DMA,
    ],
)
def _scatter_kernel(src_hbm, dst_hbm, g_hbm, zeros2_hbm, out_hbm,
                    si0, si1, di0, di1, rows0, rows1,
                    sir, dir_, rowsr, acc_sp, gsem0, gsem1, ssem0, ssem1):
    c = lax.axis_index("c")
    s = lax.axis_index("s")
    wid = c * NS + s
    ebase = wid * EPT

    def _for_node_pieces(fn_piece):
        @pl.when(s < NS - 1)
        def _():
            for p in range(4):
                fn_piece(s * NODE_A + p * CHUNK, CHUNK)
            fn_piece(s * NODE_A + 4 * CHUNK, NODE_A - 4 * CHUNK)

        @pl.when(s == NS - 1)
        def _():
            for p in range(5):
                fn_piece((NS - 1) * NODE_A + p * CHUNK, CHUNK)

    pltpu.sync_copy(zeros2_hbm, rows0)
    _for_node_pieces(lambda nstart, sz: pltpu.sync_copy(
        rows0.at[pl.ds(0, sz), :], acc_sp.at[pl.ds(nstart, sz), :]))
    plsc.subcore_barrier()

    bufs = ((si0, di0, rows0, gsem0, ssem0),
            (si1, di1, rows1, gsem1, ssem1))

    def load_idx(ci, b):
        si, di, _, _, _ = bufs[b]
        off = ebase + ci * CHUNK
        pltpu.sync_copy(src_hbm.at[pl.ds(off, CHUNK)], si)
        pltpu.sync_copy(dst_hbm.at[pl.ds(off, CHUNK)], di)

    def start_gather(b):
        si, _, rows, gsem, _ = bufs[b]
        pltpu.async_copy(g_hbm.at[si], rows, gsem)

    load_idx(0, 0)
    start_gather(0)
    load_idx(1, 1)

    @pl.loop(0, NFULL // 2)
    def _(j):
        for b in range(2):
            si, di, rows, gsem, ssem = bufs[b]
            ci = 2 * j + b
            pltpu.make_async_copy(g_hbm.at[si], rows, gsem).wait()
            pltpu.async_copy(rows, acc_sp.at[di], ssem, add=True)

            @pl.when(ci + 1 < NFULL)
            def _():
                start_gather(1 - b)

            pltpu.make_async_copy(rows, acc_sp.at[di], ssem).wait()

            @pl.when(ci + 2 < NFULL)
            def _():
                load_idx(ci + 2, b)

    off = ebase + NFULL * CHUNK
    pltpu.sync_copy(src_hbm.at[pl.ds(off, REMC)], sir)
    pltpu.sync_copy(dst_hbm.at[pl.ds(off, REMC)], dir_)
    pltpu.async_copy(g_hbm.at[sir], rowsr, gsem0).wait()
    pltpu.sync_copy(rowsr, acc_sp.at[dir_], add=True)

    plsc.subcore_barrier()

    def _writeout(nstart, sz):
        pltpu.sync_copy(acc_sp.at[pl.ds(nstart, sz), :],
                        rows0.at[pl.ds(0, sz), :])
        pltpu.sync_copy(rows0.at[pl.ds(0, sz), :],
                        out_hbm.at[c, pl.ds(nstart, sz), :])

    _for_node_pieces(_writeout)


_BLK = 1000


def _dense_body(x2_ref, degp_ref, wcat_ref, bg_ref, bt_ref,
                g_ref, base_ref, dinv_ref):
    hu = jnp.dot(x2_ref[...], wcat_ref[...],
                 preferred_element_type=jnp.float32)
    h = hu[:, :C_OUT]
    tmp = hu[:, C_OUT:]
    deg = degp_ref[:, 0:1] + degp_ref[:, 1:2] + 1.0
    dinv = lax.rsqrt(deg)
    g_ref[...] = h * dinv
    base_ref[...] = h * (dinv * dinv) + bg_ref[...] + tmp + bt_ref[...]
    dinv_ref[...] = dinv


def _dense_call(x2, degp_t, wcat, bg, bt):
    return pl.pallas_call(
        _dense_body,
        grid=(N // _BLK,),
        in_specs=[
            pl.BlockSpec((_BLK, C_IN * KT), lambda i: (i, 0)),
            pl.BlockSpec((_BLK, NC), lambda i: (i, 0)),
            pl.BlockSpec((C_IN * KT, 2 * C_OUT), lambda i: (0, 0)),
            pl.BlockSpec((1, C_OUT), lambda i: (0, 0)),
            pl.BlockSpec((1, C_OUT), lambda i: (0, 0)),
        ],
        out_specs=[
            pl.BlockSpec((_BLK, C_OUT), lambda i: (i, 0)),
            pl.BlockSpec((_BLK, C_OUT), lambda i: (i, 0)),
            pl.BlockSpec((_BLK, 1), lambda i: (i, 0)),
        ],
        out_shape=[
            jax.ShapeDtypeStruct((N, C_OUT), jnp.float32),
            jax.ShapeDtypeStruct((N, C_OUT), jnp.float32),
            jax.ShapeDtypeStruct((N, 1), jnp.float32),
        ],
    )(x2, degp_t, wcat, bg, bt)


def _combine_body(p0_ref, p1_ref, dinv_ref, base_ref, out_ref):
    out_ref[...] = (dinv_ref[...] * (p0_ref[...] + p1_ref[...])
                    + base_ref[...])


def _combine_call(p0, p1, dinv, base):
    return pl.pallas_call(
        _combine_body,
        grid=(N // _BLK,),
        in_specs=[
            pl.BlockSpec((_BLK, C_OUT), lambda i: (i, 0)),
            pl.BlockSpec((_BLK, C_OUT), lambda i: (i, 0)),
            pl.BlockSpec((_BLK, 1), lambda i: (i, 0)),
            pl.BlockSpec((_BLK, C_OUT), lambda i: (i, 0)),
        ],
        out_specs=pl.BlockSpec((_BLK, C_OUT), lambda i: (i, 0)),
        out_shape=jax.ShapeDtypeStruct((N, C_OUT), jnp.float32),
    )(p0, p1, dinv, base)


def kernel(x, edge_index, W_gcn, b_gcn, W_t, b_t):
    x2 = x.reshape(N, C_IN * KT)
    w3 = jnp.repeat(W_gcn, KT, axis=0)
    w2 = W_t.transpose(1, 2, 0).reshape(C_IN * KT, C_OUT)
    wcat = jnp.concatenate([w3, w2], axis=1)

    src = edge_index[0]
    dst = edge_index[1]

    ones1 = jnp.ones((CHUNK,), jnp.float32)
    zeros1 = jnp.zeros((NODE_B,), jnp.float32)
    zeros2 = jnp.zeros((CHUNK, C_OUT), jnp.float32)

    degp = _deg_kernel(dst, ones1, zeros1).reshape(NC, N)
    g, base, dinv = _dense_call(x2, degp.T, wcat,
                                b_gcn.reshape(1, C_OUT),
                                b_t.reshape(1, C_OUT))
    part = _scatter_kernel(src, dst, g, zeros2)
    return _combine_call(part[0], part[1], dinv, base)
